# baseline (speedup 1.0000x reference)
import functools

import jax
import jax.numpy as jnp
from jax import lax
from jax.experimental import pallas as pl
from jax.experimental.pallas import tpu as pltpu
from jax.experimental.pallas import tpu_sc as plsc

K = 10
ALPHA = 0.1
BETA = 0.9
GAMMA = BETA ** K + ALPHA * sum(BETA ** i for i in range(K))

N = 10000
E = 320000
D = 128
DH = 64
NC = 2
NS = 16
RT = 640
NP = NS * RT
ET = 20480
ECH = ET // 128
GCH = 16
EG = ECH // GCH
EP = NS * ET
NB = 4
HB = 6

_f32 = jnp.float32
_i32 = jnp.int32


def _splat16(i):
    return jnp.full((16,), i, dtype=_i32)


def _body(xh, rows, cols, out, uh, U, ACC, DEG,
          rowbA, rowbB, colbA, colbB, gb0, gb1, gb2, gb3,
          dis2t, dinvt, dist, *sems):
    gb = (gb0, gb1, gb2, gb3)
    rowb = (rowbA, rowbB)
    colb = (colbA, colbB)
    semg = sems[:NB]
    semsc = sems[NB:2 * NB]
    semi = sems[2 * NB:]
    c = lax.axis_index("c")
    s = lax.axis_index("s")
    rbase = s * RT

    def _zfill(i, carry):
        dist[pl.ds(i * 16, 16)] = jnp.zeros((16,), _f32)
        return carry
    lax.fori_loop(0, RT // 16, _zfill, 0)
    pltpu.sync_copy(dist, DEG.at[pl.ds(rbase, RT)])
    def _ofill(i, carry):
        dinvt[pl.ds(i * 16, 16)] = jnp.ones((16,), _f32)
        return carry
    lax.fori_loop(0, 8, _ofill, 0)
    plsc.subcore_barrier()

    sc_descs = [[], []]
    idx_desc = [None, None]
    pltpu.sync_copy(rows.at[s, pl.ds(0, GCH)], rowb[0])
    for g in range(EG):
        p = g & 1
        if idx_desc[p] is not None:
            idx_desc[p].wait()
            idx_desc[p] = None
        if g + 1 < EG:
            for d in sc_descs[1 - p]:
                d.wait()
            sc_descs[1 - p] = []
            idx_desc[1 - p] = pltpu.async_copy(
                rows.at[s, pl.ds((g + 1) * GCH, GCH)], rowb[1 - p],
                semi[1 - p])
        for j in range(GCH):
            sc_descs[p].append(pltpu.async_copy(
                dinvt.at[pl.ds(0, 128)], DEG.at[rowb[p].at[j]], semsc[p],
                add=True))
    for pp in range(2):
        for d in sc_descs[pp]:
            d.wait()
    plsc.subcore_barrier()

    pltpu.sync_copy(DEG.at[pl.ds(rbase, RT)], dist)
    def _normi(i, carry):
        sl = pl.ds(i * 16, 16)
        d = dist[sl] + 1.0
        b = lax.bitcast_convert_type(d, _i32)
        b = jnp.int32(0x5F3759DF) - lax.shift_right_arithmetic(b, 1)
        y = lax.bitcast_convert_type(b, _f32)
        y = y * (1.5 - 0.5 * d * y * y)
        y = y * (1.5 - 0.5 * d * y * y)
        y = y * (1.5 - 0.5 * d * y * y)
        y = y * (1.5 - 0.5 * d * y * y)
        dist[sl] = y
        dis2t[sl] = BETA * y * y
        dinvt[sl] = d * y * (1.0 / GAMMA)
        return carry
    lax.fori_loop(0, RT // 16, _normi, 0)

    for ch in range(RT // 128):
        r0 = rbase + ch * 128
        pltpu.sync_copy(xh.at[c, pl.ds(r0, 128)], gb0)
        def _initr(r, carry):
            sv = plsc.load_gather(dist, [_splat16(ch * 128 + r)])
            for f in range(DH // 16):
                sl = pl.ds(f * 16, 16)
                gb0[r, sl] = sv * gb0[r, sl]
            return carry
        lax.fori_loop(0, 128, _initr, 0)
        w1 = pltpu.async_copy(gb0, U.at[pl.ds(r0, 128)], semg[0])
        w2 = pltpu.async_copy(gb0, ACC.at[pl.ds(r0, 128)], semg[1])
        w3 = pltpu.async_copy(gb0, uh.at[c, pl.ds(r0, 128)], semg[2])
        w1.wait()
        w2.wait()
        w3.wait()
    plsc.subcore_barrier()

    def _gather_scatter():
        pltpu.sync_copy(rows.at[s, pl.ds(0, GCH)], rowb[0])
        pltpu.sync_copy(cols.at[s, pl.ds(0, GCH)], colb[0])
        dg = [None] * NB
        dsc = [None] * NB
        didx = [None, None]

        def _start_gather(nj):
            g, jj, p = nj // GCH, nj % GCH, (nj // GCH) & 1
            if jj == 0 and didx[p] is not None:
                for d in didx[p]:
                    d.wait()
                didx[p] = None
            src = uh.at[c].at[colb[p].at[jj]] if jj < HB \
                else U.at[colb[p].at[jj]]
            return pltpu.async_copy(src, gb[nj % NB], semg[nj % NB])

        for j in range(NB - 1):
            dg[j] = _start_gather(j)
        for j in range(ECH):
            b = j % NB
            g, jj, p = j // GCH, j % GCH, (j // GCH) & 1
            if jj == 4 and g + 1 < EG:
                np_ = 1 - p
                didx[np_] = (
                    pltpu.async_copy(rows.at[s, pl.ds((g + 1) * GCH, GCH)],
                                     rowb[np_], semi[np_]),
                    pltpu.async_copy(cols.at[s, pl.ds((g + 1) * GCH, GCH)],
                                     colb[np_], semi[np_]),
                )
            dg[b].wait()
            dsc[b] = pltpu.async_copy(gb[b], ACC.at[rowb[p].at[jj]],
                                      semsc[b], add=True)
            nj = j + NB - 1
            if nj < ECH:
                nb_ = nj % NB
                if dsc[nb_] is not None:
                    dsc[nb_].wait()
                dg[nb_] = _start_gather(nj)
        for j in range(NB):
            d = dsc[(ECH - 1 - j) % NB]
            if d is not None:
                d.wait()
        plsc.subcore_barrier()

    accb = (gb0, gb2)
    xb = (gb1, gb3)

    def _update_rows(ch, ab, xbuf, scale_out):
        def _updr(r, carry):
            sv = plsc.load_gather(dis2t, [_splat16(ch * 128 + r)])
            av = plsc.load_gather(dist, [_splat16(ch * 128 + r)]) * ALPHA
            if scale_out:
                gv = plsc.load_gather(dinvt, [_splat16(ch * 128 + r)])
            for f in range(DH // 16):
                sl = pl.ds(f * 16, 16)
                un = sv * ab[r, sl] + av * xbuf[r, sl]
                ab[r, sl] = un * gv if scale_out else un
            return carry
        lax.fori_loop(0, 128, _updr, 0)

    def _update_phase(scale_out):
        NCH = RT // 128
        rd = [None, None]
        wr = [None, None]

        def _start_reads(ch):
            sl_ = ch % 2
            r0 = rbase + ch * 128
            rd[sl_] = (
                pltpu.async_copy(ACC.at[pl.ds(r0, 128)], accb[sl_],
                                 semg[sl_]),
                pltpu.async_copy(xh.at[c, pl.ds(r0, 128)], xb[sl_],
                                 semg[2 + sl_]),
            )

        _start_reads(0)
        for ch in range(NCH):
            sl_ = ch % 2
            r0 = rbase + ch * 128
            for d in rd[sl_]:
                d.wait()
            _update_rows(ch, accb[sl_], xb[sl_], scale_out)
            if scale_out:
                wr[sl_] = (
                    pltpu.async_copy(accb[sl_], out.at[c, pl.ds(r0, 128)],
                                     semsc[sl_]),
                )
            else:
                wr[sl_] = (
                    pltpu.async_copy(accb[sl_], U.at[pl.ds(r0, 128)],
                                     semsc[sl_]),
                    pltpu.async_copy(accb[sl_], ACC.at[pl.ds(r0, 128)],
                                     semsc[2 + sl_]),
                    pltpu.async_copy(accb[sl_], uh.at[c, pl.ds(r0, 128)],
                                     semi[sl_]),
                )
            if ch + 1 < NCH:
                osl = (ch + 1) % 2
                if wr[osl] is not None:
                    for w in wr[osl]:
                        w.wait()
                    wr[osl] = None
                _start_reads(ch + 1)
        for q in range(2):
            if wr[q] is not None:
                for w in wr[q]:
                    w.wait()

    def _round(k, carry):
        _gather_scatter()
        _update_phase(scale_out=False)
        plsc.subcore_barrier()
        return carry
    lax.fori_loop(0, K - 1, _round, 0)

    _gather_scatter()
    _update_phase(scale_out=True)


@functools.partial(
    pl.kernel,
    out_type=(jax.ShapeDtypeStruct((NC, NP, DH), _f32),
              jax.ShapeDtypeStruct((NC, NP, DH), _f32)),
    mesh=plsc.VectorSubcoreMesh(
        core_axis_name="c", subcore_axis_name="s",
        num_cores=NC, num_subcores=NS),
    scratch_types=[
        pltpu.VMEM_SHARED((NP, DH), _f32),
        pltpu.VMEM_SHARED((NP, DH), _f32),
        pltpu.VMEM_SHARED((NP,), _f32),
        pltpu.VMEM((GCH, 128), _i32),
        pltpu.VMEM((GCH, 128), _i32),
        pltpu.VMEM((GCH, 128), _i32),
        pltpu.VMEM((GCH, 128), _i32),
        pltpu.VMEM((128, DH), _f32),
        pltpu.VMEM((128, DH), _f32),
        pltpu.VMEM((128, DH), _f32),
        pltpu.VMEM((128, DH), _f32),
        pltpu.VMEM((RT,), _f32),
        pltpu.VMEM((RT,), _f32),
        pltpu.VMEM((RT,), _f32),
        pltpu.SemaphoreType.DMA,
        pltpu.SemaphoreType.DMA,
        pltpu.SemaphoreType.DMA,
        pltpu.SemaphoreType.DMA,
        pltpu.SemaphoreType.DMA,
        pltpu.SemaphoreType.DMA,
        pltpu.SemaphoreType.DMA,
        pltpu.SemaphoreType.DMA,
        pltpu.SemaphoreType.DMA,
        pltpu.SemaphoreType.DMA,
    ],
    compiler_params=pltpu.CompilerParams(
        needs_layout_passes=False, use_tc_tiling_on_sc=False),
)
def _propagate(xh, rows, cols, out, uh, *scratch):
    _body(xh, rows, cols, out, uh, *scratch)


def kernel(x, edge_index):
    row = edge_index[0].astype(_i32)
    col = edge_index[1].astype(_i32)
    rows_p = jnp.full((EP,), N, _i32).at[:E].set(row).reshape(NS, ECH, 128)
    cols_p = jnp.full((EP,), N, _i32).at[:E].set(col).reshape(NS, ECH, 128)
    xh = jnp.zeros((NC, NP, DH), _f32).at[:, :N, :].set(
        x.reshape(N, NC, DH).transpose(1, 0, 2))
    out, _ = _propagate(xh, rows_p, cols_p)
    return out[:, :N, :].transpose(1, 0, 2).reshape(N, D)


# speedup vs baseline: 1.3541x; 1.3541x over previous
import functools

import jax
import jax.numpy as jnp
from jax import lax
from jax.experimental import pallas as pl
from jax.experimental.pallas import tpu as pltpu
from jax.experimental.pallas import tpu_sc as plsc

K = 10
ALPHA = 0.1
BETA = 0.9
GAMMA = BETA ** K + ALPHA * sum(BETA ** i for i in range(K))

N = 10000
E = 320000
D = 128
DH = 64
NC = 2
NS = 16
RT = 640
NP = NS * RT
ET = 20480
ECH = ET // 128
GCH = 16
EG = ECH // GCH
EP = NS * ET
NB = 4
HB = 0

_f32 = jnp.float32
_i32 = jnp.int32


def _splat16(i):
    return jnp.full((16,), i, dtype=_i32)


def _body(xh, rows, cols, out, uh, U, ACC, DEG,
          rowbA, rowbB, colbA, colbB, gb0, gb1, gb2, gb3,
          dis2t, dinvt, dist, *sems):
    gb = (gb0, gb1, gb2, gb3)
    rowb = (rowbA, rowbB)
    colb = (colbA, colbB)
    semg = sems[:NB]
    semsc = sems[NB:2 * NB]
    semi = sems[2 * NB:]
    c = lax.axis_index("c")
    s = lax.axis_index("s")
    rbase = s * RT

    def _zfill(i, carry):
        dist[pl.ds(i * 16, 16)] = jnp.zeros((16,), _f32)
        return carry
    lax.fori_loop(0, RT // 16, _zfill, 0)
    pltpu.sync_copy(dist, DEG.at[pl.ds(rbase, RT)])
    def _ofill(i, carry):
        dinvt[pl.ds(i * 16, 16)] = jnp.ones((16,), _f32)
        return carry
    lax.fori_loop(0, 8, _ofill, 0)
    plsc.subcore_barrier()

    sc_descs = [[], []]
    idx_desc = [None, None]
    pltpu.sync_copy(rows.at[s, pl.ds(0, GCH)], rowb[0])
    for g in range(EG):
        p = g & 1
        if idx_desc[p] is not None:
            idx_desc[p].wait()
            idx_desc[p] = None
        if g + 1 < EG:
            for d in sc_descs[1 - p]:
                d.wait()
            sc_descs[1 - p] = []
            idx_desc[1 - p] = pltpu.async_copy(
                rows.at[s, pl.ds((g + 1) * GCH, GCH)], rowb[1 - p],
                semi[1 - p])
        for j in range(GCH):
            sc_descs[p].append(pltpu.async_copy(
                dinvt.at[pl.ds(0, 128)], DEG.at[rowb[p].at[j]], semsc[p],
                add=True))
    for pp in range(2):
        for d in sc_descs[pp]:
            d.wait()
    plsc.subcore_barrier()

    pltpu.sync_copy(DEG.at[pl.ds(rbase, RT)], dist)
    def _normi(i, carry):
        sl = pl.ds(i * 16, 16)
        d = dist[sl] + 1.0
        b = lax.bitcast_convert_type(d, _i32)
        b = jnp.int32(0x5F3759DF) - lax.shift_right_arithmetic(b, 1)
        y = lax.bitcast_convert_type(b, _f32)
        y = y * (1.5 - 0.5 * d * y * y)
        y = y * (1.5 - 0.5 * d * y * y)
        y = y * (1.5 - 0.5 * d * y * y)
        y = y * (1.5 - 0.5 * d * y * y)
        dist[sl] = y
        dis2t[sl] = BETA * y * y
        dinvt[sl] = d * y * (1.0 / GAMMA)
        return carry
    lax.fori_loop(0, RT // 16, _normi, 0)

    for ch in range(RT // 128):
        r0 = rbase + ch * 128
        pltpu.sync_copy(xh.at[c, pl.ds(r0, 128)], gb0)
        def _initr(r, carry):
            sv = plsc.load_gather(dist, [_splat16(ch * 128 + r)])
            for f in range(DH // 16):
                sl = pl.ds(f * 16, 16)
                gb0[r, sl] = sv * gb0[r, sl]
            return carry
        lax.fori_loop(0, 128, _initr, 0)
        w1 = pltpu.async_copy(gb0, U.at[pl.ds(r0, 128)], semg[0])
        w2 = pltpu.async_copy(gb0, ACC.at[pl.ds(r0, 128)], semg[1])
        w3 = pltpu.async_copy(gb0, uh.at[c, pl.ds(r0, 128)], semg[2])
        w1.wait()
        w2.wait()
        w3.wait()
    plsc.subcore_barrier()

    def _gather_scatter():
        pltpu.sync_copy(rows.at[s, pl.ds(0, GCH)], rowb[0])
        pltpu.sync_copy(cols.at[s, pl.ds(0, GCH)], colb[0])
        dg = [None] * NB
        dsc = [None] * NB
        didx = [None, None]

        def _start_gather(nj):
            g, jj, p = nj // GCH, nj % GCH, (nj // GCH) & 1
            if jj == 0 and didx[p] is not None:
                for d in didx[p]:
                    d.wait()
                didx[p] = None
            src = uh.at[c].at[colb[p].at[jj]] if jj < HB \
                else U.at[colb[p].at[jj]]
            return pltpu.async_copy(src, gb[nj % NB], semg[nj % NB])

        for j in range(NB - 1):
            dg[j] = _start_gather(j)
        for j in range(ECH):
            b = j % NB
            g, jj, p = j // GCH, j % GCH, (j // GCH) & 1
            if jj == 4 and g + 1 < EG:
                np_ = 1 - p
                didx[np_] = (
                    pltpu.async_copy(rows.at[s, pl.ds((g + 1) * GCH, GCH)],
                                     rowb[np_], semi[np_]),
                    pltpu.async_copy(cols.at[s, pl.ds((g + 1) * GCH, GCH)],
                                     colb[np_], semi[np_]),
                )
            dg[b].wait()
            dsc[b] = pltpu.async_copy(gb[b], ACC.at[rowb[p].at[jj]],
                                      semsc[b], add=True)
            nj = j + NB - 1
            if nj < ECH:
                nb_ = nj % NB
                if dsc[nb_] is not None:
                    dsc[nb_].wait()
                dg[nb_] = _start_gather(nj)
        for j in range(NB):
            d = dsc[(ECH - 1 - j) % NB]
            if d is not None:
                d.wait()
        plsc.subcore_barrier()

    accb = (gb0, gb2)
    xb = (gb1, gb3)

    def _update_rows(ch, ab, xbuf, scale_out):
        def _updr(r, carry):
            sv = plsc.load_gather(dis2t, [_splat16(ch * 128 + r)])
            av = plsc.load_gather(dist, [_splat16(ch * 128 + r)]) * ALPHA
            if scale_out:
                gv = plsc.load_gather(dinvt, [_splat16(ch * 128 + r)])
            for f in range(DH // 16):
                sl = pl.ds(f * 16, 16)
                un = sv * ab[r, sl] + av * xbuf[r, sl]
                ab[r, sl] = un * gv if scale_out else un
            return carry
        lax.fori_loop(0, 128, _updr, 0)

    def _update_phase(scale_out):
        NCH = RT // 128
        rd = [None, None]
        wr = [None, None]

        def _start_reads(ch):
            sl_ = ch % 2
            r0 = rbase + ch * 128
            rd[sl_] = (
                pltpu.async_copy(ACC.at[pl.ds(r0, 128)], accb[sl_],
                                 semg[sl_]),
                pltpu.async_copy(xh.at[c, pl.ds(r0, 128)], xb[sl_],
                                 semg[2 + sl_]),
            )

        _start_reads(0)
        for ch in range(NCH):
            sl_ = ch % 2
            r0 = rbase + ch * 128
            for d in rd[sl_]:
                d.wait()
            _update_rows(ch, accb[sl_], xb[sl_], scale_out)
            if scale_out:
                wr[sl_] = (
                    pltpu.async_copy(accb[sl_], out.at[c, pl.ds(r0, 128)],
                                     semsc[sl_]),
                )
            else:
                wr[sl_] = (
                    pltpu.async_copy(accb[sl_], U.at[pl.ds(r0, 128)],
                                     semsc[sl_]),
                    pltpu.async_copy(accb[sl_], ACC.at[pl.ds(r0, 128)],
                                     semsc[2 + sl_]),
                    pltpu.async_copy(accb[sl_], uh.at[c, pl.ds(r0, 128)],
                                     semi[sl_]),
                )
            if ch + 1 < NCH:
                osl = (ch + 1) % 2
                if wr[osl] is not None:
                    for w in wr[osl]:
                        w.wait()
                    wr[osl] = None
                _start_reads(ch + 1)
        for q in range(2):
            if wr[q] is not None:
                for w in wr[q]:
                    w.wait()

    def _round(k, carry):
        _gather_scatter()
        _update_phase(scale_out=False)
        plsc.subcore_barrier()
        return carry
    lax.fori_loop(0, K - 1, _round, 0)

    _gather_scatter()
    _update_phase(scale_out=True)


@functools.partial(
    pl.kernel,
    out_type=(jax.ShapeDtypeStruct((NC, NP, DH), _f32),
              jax.ShapeDtypeStruct((NC, NP, DH), _f32)),
    mesh=plsc.VectorSubcoreMesh(
        core_axis_name="c", subcore_axis_name="s",
        num_cores=NC, num_subcores=NS),
    scratch_types=[
        pltpu.VMEM_SHARED((NP, DH), _f32),
        pltpu.VMEM_SHARED((NP, DH), _f32),
        pltpu.VMEM_SHARED((NP,), _f32),
        pltpu.VMEM((GCH, 128), _i32),
        pltpu.VMEM((GCH, 128), _i32),
        pltpu.VMEM((GCH, 128), _i32),
        pltpu.VMEM((GCH, 128), _i32),
        pltpu.VMEM((128, DH), _f32),
        pltpu.VMEM((128, DH), _f32),
        pltpu.VMEM((128, DH), _f32),
        pltpu.VMEM((128, DH), _f32),
        pltpu.VMEM((RT,), _f32),
        pltpu.VMEM((RT,), _f32),
        pltpu.VMEM((RT,), _f32),
        pltpu.SemaphoreType.DMA,
        pltpu.SemaphoreType.DMA,
        pltpu.SemaphoreType.DMA,
        pltpu.SemaphoreType.DMA,
        pltpu.SemaphoreType.DMA,
        pltpu.SemaphoreType.DMA,
        pltpu.SemaphoreType.DMA,
        pltpu.SemaphoreType.DMA,
        pltpu.SemaphoreType.DMA,
        pltpu.SemaphoreType.DMA,
    ],
    compiler_params=pltpu.CompilerParams(
        needs_layout_passes=False, use_tc_tiling_on_sc=False),
)
def _propagate(xh, rows, cols, out, uh, *scratch):
    _body(xh, rows, cols, out, uh, *scratch)


def kernel(x, edge_index):
    row = edge_index[0].astype(_i32)
    col = edge_index[1].astype(_i32)
    rows_p = jnp.full((EP,), N, _i32).at[:E].set(row).reshape(NS, ECH, 128)
    cols_p = jnp.full((EP,), N, _i32).at[:E].set(col).reshape(NS, ECH, 128)
    xh = jnp.zeros((NC, NP, DH), _f32).at[:, :N, :].set(
        x.reshape(N, NC, DH).transpose(1, 0, 2))
    out, _ = _propagate(xh, rows_p, cols_p)
    return out[:, :N, :].transpose(1, 0, 2).reshape(N, D)


# speedup vs baseline: 1.4894x; 1.1000x over previous
import functools

import jax
import jax.numpy as jnp
from jax import lax
from jax.experimental import pallas as pl
from jax.experimental.pallas import tpu as pltpu
from jax.experimental.pallas import tpu_sc as plsc

K = 10
ALPHA = 0.1
BETA = 0.9
GAMMA = BETA ** K + ALPHA * sum(BETA ** i for i in range(K))

N = 10000
E = 320000
D = 128
DH = 64
NC = 2
NS = 16
RT = 640
NP = NS * RT
ET = 20480
ECH = ET // 128
GCH = 16
EG = ECH // GCH
EP = NS * ET
NB = 4

_f32 = jnp.float32
_i32 = jnp.int32


def _splat16(i):
    return jnp.full((16,), i, dtype=_i32)


def _body(xh, rows, cols, out, U, ACC, DEG,
          rowbA, rowbB, colbA, colbB, gb0, gb1, gb2, gb3,
          dis2t, dinvt, dist, *sems):
    gb = (gb0, gb1, gb2, gb3)
    rowb = (rowbA, rowbB)
    colb = (colbA, colbB)
    semg = sems[:NB]
    semsc = sems[NB:2 * NB]
    semi = sems[2 * NB:]
    c = lax.axis_index("c")
    s = lax.axis_index("s")
    rbase = s * RT

    def _zfill(i, carry):
        dist[pl.ds(i * 16, 16)] = jnp.zeros((16,), _f32)
        return carry
    lax.fori_loop(0, RT // 16, _zfill, 0)
    pltpu.sync_copy(dist, DEG.at[pl.ds(rbase, RT)])
    def _ofill(i, carry):
        dinvt[pl.ds(i * 16, 16)] = jnp.ones((16,), _f32)
        return carry
    lax.fori_loop(0, 8, _ofill, 0)
    plsc.subcore_barrier()

    sc_descs = [[], []]
    idx_desc = [None, None]
    pltpu.sync_copy(rows.at[s, pl.ds(0, GCH)], rowb[0])
    for g in range(EG):
        p = g & 1
        if idx_desc[p] is not None:
            idx_desc[p].wait()
            idx_desc[p] = None
        if g + 1 < EG:
            for d in sc_descs[1 - p]:
                d.wait()
            sc_descs[1 - p] = []
            idx_desc[1 - p] = pltpu.async_copy(
                rows.at[s, pl.ds((g + 1) * GCH, GCH)], rowb[1 - p],
                semi[1 - p])
        for j in range(GCH):
            sc_descs[p].append(pltpu.async_copy(
                dinvt.at[pl.ds(0, 128)], DEG.at[rowb[p].at[j]], semsc[p],
                add=True))
    for pp in range(2):
        for d in sc_descs[pp]:
            d.wait()
    plsc.subcore_barrier()

    pltpu.sync_copy(DEG.at[pl.ds(rbase, RT)], dist)
    def _normi(i, carry):
        sl = pl.ds(i * 16, 16)
        d = dist[sl] + 1.0
        b = lax.bitcast_convert_type(d, _i32)
        b = jnp.int32(0x5F3759DF) - lax.shift_right_arithmetic(b, 1)
        y = lax.bitcast_convert_type(b, _f32)
        y = y * (1.5 - 0.5 * d * y * y)
        y = y * (1.5 - 0.5 * d * y * y)
        y = y * (1.5 - 0.5 * d * y * y)
        y = y * (1.5 - 0.5 * d * y * y)
        dist[sl] = y
        dis2t[sl] = BETA * y * y
        dinvt[sl] = d * y * (1.0 / GAMMA)
        return carry
    lax.fori_loop(0, RT // 16, _normi, 0)

    for ch in range(RT // 128):
        r0 = rbase + ch * 128
        pltpu.sync_copy(xh.at[c, pl.ds(r0, 128)], gb0)
        @plsc.parallel_loop(0, 128, 1, unroll=4)
        def _initr(r):
            sv = plsc.load_gather(dist, [_splat16(ch * 128 + r)])
            for f in range(DH // 16):
                sl = pl.ds(f * 16, 16)
                gb0[r, sl] = sv * gb0[r, sl]
        w1 = pltpu.async_copy(gb0, U.at[pl.ds(r0, 128)], semg[0])
        w2 = pltpu.async_copy(gb0, ACC.at[pl.ds(r0, 128)], semg[1])
        w1.wait()
        w2.wait()
    plsc.subcore_barrier()

    def _gather_scatter():
        pltpu.sync_copy(rows.at[s, pl.ds(0, GCH)], rowb[0])
        pltpu.sync_copy(cols.at[s, pl.ds(0, GCH)], colb[0])
        dg = [None] * NB
        dsc = [None] * NB
        didx = [None, None]

        def _start_gather(nj):
            g, jj, p = nj // GCH, nj % GCH, (nj // GCH) & 1
            if jj == 0 and didx[p] is not None:
                for d in didx[p]:
                    d.wait()
                didx[p] = None
            return pltpu.async_copy(U.at[colb[p].at[jj]], gb[nj % NB],
                                    semg[nj % NB])

        for j in range(NB - 1):
            dg[j] = _start_gather(j)
        for j in range(ECH):
            b = j % NB
            g, jj, p = j // GCH, j % GCH, (j // GCH) & 1
            if jj == 4 and g + 1 < EG:
                np_ = 1 - p
                didx[np_] = (
                    pltpu.async_copy(rows.at[s, pl.ds((g + 1) * GCH, GCH)],
                                     rowb[np_], semi[np_]),
                    pltpu.async_copy(cols.at[s, pl.ds((g + 1) * GCH, GCH)],
                                     colb[np_], semi[np_]),
                )
            dg[b].wait()
            dsc[b] = pltpu.async_copy(gb[b], ACC.at[rowb[p].at[jj]],
                                      semsc[b], add=True)
            nj = j + NB - 1
            if nj < ECH:
                nb_ = nj % NB
                if dsc[nb_] is not None:
                    dsc[nb_].wait()
                dg[nb_] = _start_gather(nj)
        for j in range(NB):
            d = dsc[(ECH - 1 - j) % NB]
            if d is not None:
                d.wait()
        plsc.subcore_barrier()

    accb = (gb0, gb2)
    xb = (gb1, gb3)

    def _update_rows(ch, ab, xbuf, scale_out):
        @plsc.parallel_loop(0, 128, 1, unroll=4)
        def _updr(r):
            sv = plsc.load_gather(dis2t, [_splat16(ch * 128 + r)])
            av = plsc.load_gather(dist, [_splat16(ch * 128 + r)]) * ALPHA
            if scale_out:
                gv = plsc.load_gather(dinvt, [_splat16(ch * 128 + r)])
            for f in range(DH // 16):
                sl = pl.ds(f * 16, 16)
                un = sv * ab[r, sl] + av * xbuf[r, sl]
                ab[r, sl] = un * gv if scale_out else un

    def _update_phase(scale_out):
        NCH = RT // 128
        rd = [None, None]
        wr = [None, None]

        def _start_reads(ch):
            sl_ = ch % 2
            r0 = rbase + ch * 128
            rd[sl_] = (
                pltpu.async_copy(ACC.at[pl.ds(r0, 128)], accb[sl_],
                                 semg[sl_]),
                pltpu.async_copy(xh.at[c, pl.ds(r0, 128)], xb[sl_],
                                 semg[2 + sl_]),
            )

        _start_reads(0)
        for ch in range(NCH):
            sl_ = ch % 2
            r0 = rbase + ch * 128
            for d in rd[sl_]:
                d.wait()
            _update_rows(ch, accb[sl_], xb[sl_], scale_out)
            if scale_out:
                wr[sl_] = (
                    pltpu.async_copy(accb[sl_], out.at[c, pl.ds(r0, 128)],
                                     semsc[sl_]),
                )
            else:
                wr[sl_] = (
                    pltpu.async_copy(accb[sl_], U.at[pl.ds(r0, 128)],
                                     semsc[sl_]),
                    pltpu.async_copy(accb[sl_], ACC.at[pl.ds(r0, 128)],
                                     semsc[2 + sl_]),
                )
            if ch + 1 < NCH:
                osl = (ch + 1) % 2
                if wr[osl] is not None:
                    for w in wr[osl]:
                        w.wait()
                    wr[osl] = None
                _start_reads(ch + 1)
        for q in range(2):
            if wr[q] is not None:
                for w in wr[q]:
                    w.wait()

    def _round(k, carry):
        _gather_scatter()
        _update_phase(scale_out=False)
        plsc.subcore_barrier()
        return carry
    lax.fori_loop(0, K - 1, _round, 0)

    _gather_scatter()
    _update_phase(scale_out=True)


@functools.partial(
    pl.kernel,
    out_type=jax.ShapeDtypeStruct((NC, NP, DH), _f32),
    mesh=plsc.VectorSubcoreMesh(
        core_axis_name="c", subcore_axis_name="s",
        num_cores=NC, num_subcores=NS),
    scratch_types=[
        pltpu.VMEM_SHARED((NP, DH), _f32),
        pltpu.VMEM_SHARED((NP, DH), _f32),
        pltpu.VMEM_SHARED((NP,), _f32),
        pltpu.VMEM((GCH, 128), _i32),
        pltpu.VMEM((GCH, 128), _i32),
        pltpu.VMEM((GCH, 128), _i32),
        pltpu.VMEM((GCH, 128), _i32),
        pltpu.VMEM((128, DH), _f32),
        pltpu.VMEM((128, DH), _f32),
        pltpu.VMEM((128, DH), _f32),
        pltpu.VMEM((128, DH), _f32),
        pltpu.VMEM((RT,), _f32),
        pltpu.VMEM((RT,), _f32),
        pltpu.VMEM((RT,), _f32),
        pltpu.SemaphoreType.DMA,
        pltpu.SemaphoreType.DMA,
        pltpu.SemaphoreType.DMA,
        pltpu.SemaphoreType.DMA,
        pltpu.SemaphoreType.DMA,
        pltpu.SemaphoreType.DMA,
        pltpu.SemaphoreType.DMA,
        pltpu.SemaphoreType.DMA,
        pltpu.SemaphoreType.DMA,
        pltpu.SemaphoreType.DMA,
    ],
    compiler_params=pltpu.CompilerParams(
        needs_layout_passes=False, use_tc_tiling_on_sc=False),
)
def _propagate(xh, rows, cols, out, *scratch):
    _body(xh, rows, cols, out, *scratch)


def kernel(x, edge_index):
    row = edge_index[0].astype(_i32)
    col = edge_index[1].astype(_i32)
    rows_p = jnp.full((EP,), N, _i32).at[:E].set(row).reshape(NS, ECH, 128)
    cols_p = jnp.full((EP,), N, _i32).at[:E].set(col).reshape(NS, ECH, 128)
    xh = jnp.zeros((NC, NP, DH), _f32).at[:, :N, :].set(
        x.reshape(N, NC, DH).transpose(1, 0, 2))
    out = _propagate(xh, rows_p, cols_p)
    return out[:, :N, :].transpose(1, 0, 2).reshape(N, D)


# speedup vs baseline: 1.4901x; 1.0005x over previous
import functools

import jax
import jax.numpy as jnp
from jax import lax
from jax.experimental import pallas as pl
from jax.experimental.pallas import tpu as pltpu
from jax.experimental.pallas import tpu_sc as plsc

K = 10
ALPHA = 0.1
BETA = 0.9
GAMMA = BETA ** K + ALPHA * sum(BETA ** i for i in range(K))

N = 10000
E = 320000
D = 128
DH = 64
NC = 2
NS = 16
RT = 640
NP = NS * RT
ET = 20480
ECH = ET // 128
GCH = 16
EG = ECH // GCH
EP = NS * ET
NB = 4

_f32 = jnp.float32
_i32 = jnp.int32


def _splat16(i):
    return jnp.full((16,), i, dtype=_i32)


def _body(xh, rows, cols, out, U, ACC, DEG,
          rowbA, rowbB, colbA, colbB, gb0, gb1, gb2, gb3,
          dis2t, dinvt, dist, *sems):
    gb = (gb0, gb1, gb2, gb3)
    rowb = (rowbA, rowbB)
    colb = (colbA, colbB)
    semg = sems[:NB]
    semsc = sems[NB:2 * NB]
    semi = sems[2 * NB:]
    c = lax.axis_index("c")
    s = lax.axis_index("s")
    rbase = s * RT

    def _zfill(i, carry):
        dist[pl.ds(i * 16, 16)] = jnp.zeros((16,), _f32)
        return carry
    lax.fori_loop(0, RT // 16, _zfill, 0)
    pltpu.sync_copy(dist, DEG.at[pl.ds(rbase, RT)])
    def _ofill(i, carry):
        dinvt[pl.ds(i * 16, 16)] = jnp.ones((16,), _f32)
        return carry
    lax.fori_loop(0, 8, _ofill, 0)
    plsc.subcore_barrier()

    sc_descs = [[], []]
    idx_desc = [None, None]
    pltpu.sync_copy(rows.at[s, pl.ds(0, GCH)], rowb[0])
    for g in range(EG):
        p = g & 1
        if idx_desc[p] is not None:
            idx_desc[p].wait()
            idx_desc[p] = None
        if g + 1 < EG:
            for d in sc_descs[1 - p]:
                d.wait()
            sc_descs[1 - p] = []
            idx_desc[1 - p] = pltpu.async_copy(
                rows.at[s, pl.ds((g + 1) * GCH, GCH)], rowb[1 - p],
                semi[1 - p])
        for j in range(GCH):
            sc_descs[p].append(pltpu.async_copy(
                dinvt.at[pl.ds(0, 128)], DEG.at[rowb[p].at[j]], semsc[p],
                add=True))
    for pp in range(2):
        for d in sc_descs[pp]:
            d.wait()
    plsc.subcore_barrier()

    pltpu.sync_copy(DEG.at[pl.ds(rbase, RT)], dist)
    def _normi(i, carry):
        sl = pl.ds(i * 16, 16)
        d = dist[sl] + 1.0
        b = lax.bitcast_convert_type(d, _i32)
        b = jnp.int32(0x5F3759DF) - lax.shift_right_arithmetic(b, 1)
        y = lax.bitcast_convert_type(b, _f32)
        y = y * (1.5 - 0.5 * d * y * y)
        y = y * (1.5 - 0.5 * d * y * y)
        y = y * (1.5 - 0.5 * d * y * y)
        y = y * (1.5 - 0.5 * d * y * y)
        dist[sl] = y
        dis2t[sl] = BETA * y * y
        dinvt[sl] = d * y * (1.0 / GAMMA)
        return carry
    lax.fori_loop(0, RT // 16, _normi, 0)

    for ch in range(RT // 128):
        r0 = rbase + ch * 128
        pltpu.sync_copy(xh.at[c, pl.ds(r0, 128)], gb0)
        @plsc.parallel_loop(0, 128, 1, unroll=8)
        def _initr(r):
            sv = plsc.load_gather(dist, [_splat16(ch * 128 + r)])
            for f in range(DH // 16):
                sl = pl.ds(f * 16, 16)
                gb0[r, sl] = sv * gb0[r, sl]
        w1 = pltpu.async_copy(gb0, U.at[pl.ds(r0, 128)], semg[0])
        w2 = pltpu.async_copy(gb0, ACC.at[pl.ds(r0, 128)], semg[1])
        w1.wait()
        w2.wait()
    plsc.subcore_barrier()

    def _gather_scatter():
        pltpu.sync_copy(rows.at[s, pl.ds(0, GCH)], rowb[0])
        pltpu.sync_copy(cols.at[s, pl.ds(0, GCH)], colb[0])
        dg = [None] * NB
        dsc = [None] * NB
        didx = [None, None]

        def _start_gather(nj):
            g, jj, p = nj // GCH, nj % GCH, (nj // GCH) & 1
            if jj == 0 and didx[p] is not None:
                for d in didx[p]:
                    d.wait()
                didx[p] = None
            return pltpu.async_copy(U.at[colb[p].at[jj]], gb[nj % NB],
                                    semg[nj % NB])

        for j in range(NB - 1):
            dg[j] = _start_gather(j)
        for j in range(ECH):
            b = j % NB
            g, jj, p = j // GCH, j % GCH, (j // GCH) & 1
            if jj == 4 and g + 1 < EG:
                np_ = 1 - p
                didx[np_] = (
                    pltpu.async_copy(rows.at[s, pl.ds((g + 1) * GCH, GCH)],
                                     rowb[np_], semi[np_]),
                    pltpu.async_copy(cols.at[s, pl.ds((g + 1) * GCH, GCH)],
                                     colb[np_], semi[np_]),
                )
            dg[b].wait()
            dsc[b] = pltpu.async_copy(gb[b], ACC.at[rowb[p].at[jj]],
                                      semsc[b], add=True)
            nj = j + NB - 1
            if nj < ECH:
                nb_ = nj % NB
                if dsc[nb_] is not None:
                    dsc[nb_].wait()
                dg[nb_] = _start_gather(nj)
        for j in range(NB):
            d = dsc[(ECH - 1 - j) % NB]
            if d is not None:
                d.wait()
        plsc.subcore_barrier()

    accb = (gb0, gb2)
    xb = (gb1, gb3)

    def _update_rows(ch, ab, xbuf, scale_out):
        @plsc.parallel_loop(0, 128, 1, unroll=8)
        def _updr(r):
            sv = plsc.load_gather(dis2t, [_splat16(ch * 128 + r)])
            av = plsc.load_gather(dist, [_splat16(ch * 128 + r)]) * ALPHA
            if scale_out:
                gv = plsc.load_gather(dinvt, [_splat16(ch * 128 + r)])
            for f in range(DH // 16):
                sl = pl.ds(f * 16, 16)
                un = sv * ab[r, sl] + av * xbuf[r, sl]
                ab[r, sl] = un * gv if scale_out else un

    def _update_phase(scale_out):
        NCH = RT // 128
        rd = [None, None]
        wr = [None, None]

        def _start_reads(ch):
            sl_ = ch % 2
            r0 = rbase + ch * 128
            rd[sl_] = (
                pltpu.async_copy(ACC.at[pl.ds(r0, 128)], accb[sl_],
                                 semg[sl_]),
                pltpu.async_copy(xh.at[c, pl.ds(r0, 128)], xb[sl_],
                                 semg[2 + sl_]),
            )

        _start_reads(0)
        for ch in range(NCH):
            sl_ = ch % 2
            r0 = rbase + ch * 128
            for d in rd[sl_]:
                d.wait()
            _update_rows(ch, accb[sl_], xb[sl_], scale_out)
            if scale_out:
                wr[sl_] = (
                    pltpu.async_copy(accb[sl_], out.at[c, pl.ds(r0, 128)],
                                     semsc[sl_]),
                )
            else:
                wr[sl_] = (
                    pltpu.async_copy(accb[sl_], U.at[pl.ds(r0, 128)],
                                     semsc[sl_]),
                    pltpu.async_copy(accb[sl_], ACC.at[pl.ds(r0, 128)],
                                     semsc[2 + sl_]),
                )
            if ch + 1 < NCH:
                osl = (ch + 1) % 2
                if wr[osl] is not None:
                    for w in wr[osl]:
                        w.wait()
                    wr[osl] = None
                _start_reads(ch + 1)
        for q in range(2):
            if wr[q] is not None:
                for w in wr[q]:
                    w.wait()

    def _round(k, carry):
        _gather_scatter()
        _update_phase(scale_out=False)
        plsc.subcore_barrier()
        return carry
    lax.fori_loop(0, K - 1, _round, 0)

    _gather_scatter()
    _update_phase(scale_out=True)


@functools.partial(
    pl.kernel,
    out_type=jax.ShapeDtypeStruct((NC, NP, DH), _f32),
    mesh=plsc.VectorSubcoreMesh(
        core_axis_name="c", subcore_axis_name="s",
        num_cores=NC, num_subcores=NS),
    scratch_types=[
        pltpu.VMEM_SHARED((NP, DH), _f32),
        pltpu.VMEM_SHARED((NP, DH), _f32),
        pltpu.VMEM_SHARED((NP,), _f32),
        pltpu.VMEM((GCH, 128), _i32),
        pltpu.VMEM((GCH, 128), _i32),
        pltpu.VMEM((GCH, 128), _i32),
        pltpu.VMEM((GCH, 128), _i32),
        pltpu.VMEM((128, DH), _f32),
        pltpu.VMEM((128, DH), _f32),
        pltpu.VMEM((128, DH), _f32),
        pltpu.VMEM((128, DH), _f32),
        pltpu.VMEM((RT,), _f32),
        pltpu.VMEM((RT,), _f32),
        pltpu.VMEM((RT,), _f32),
        pltpu.SemaphoreType.DMA,
        pltpu.SemaphoreType.DMA,
        pltpu.SemaphoreType.DMA,
        pltpu.SemaphoreType.DMA,
        pltpu.SemaphoreType.DMA,
        pltpu.SemaphoreType.DMA,
        pltpu.SemaphoreType.DMA,
        pltpu.SemaphoreType.DMA,
        pltpu.SemaphoreType.DMA,
        pltpu.SemaphoreType.DMA,
    ],
    compiler_params=pltpu.CompilerParams(
        needs_layout_passes=False, use_tc_tiling_on_sc=False),
)
def _propagate(xh, rows, cols, out, *scratch):
    _body(xh, rows, cols, out, *scratch)


def kernel(x, edge_index):
    row = edge_index[0].astype(_i32)
    col = edge_index[1].astype(_i32)
    rows_p = jnp.full((EP,), N, _i32).at[:E].set(row).reshape(NS, ECH, 128)
    cols_p = jnp.full((EP,), N, _i32).at[:E].set(col).reshape(NS, ECH, 128)
    xh = jnp.zeros((NC, NP, DH), _f32).at[:, :N, :].set(
        x.reshape(N, NC, DH).transpose(1, 0, 2))
    out = _propagate(xh, rows_p, cols_p)
    return out[:, :N, :].transpose(1, 0, 2).reshape(N, D)
